# baseline (speedup 1.0000x reference)
import dataclasses
import functools

import jax
import jax.numpy as jnp
from jax import lax
from jax.experimental import pallas as pl
from jax.experimental.pallas import tpu as pltpu
from jax.experimental.pallas import tpu_sc as plsc

D = 256
AW = 200
RS = 208
CH = 128
NWT = 25
E_BLK = 256


def _sc_agg_body(xq_hbm, ids_hbm, bounds_hbm, zeros_hbm, agg_hbm,
                 staging, xbuf, idbuf, bsm):
    nq = ids_hbm.shape[0]
    wid = lax.axis_index("s") * 2 + lax.axis_index("c")
    lane = lax.iota(jnp.int32, 16)
    laneD = lane * D
    pltpu.sync_copy(bounds_hbm, bsm)

    def window(k, carry):
        win = wid * NWT + k
        winbase = win * AW
        bv = bsm[pl.ds(win, 16)]
        a = bv[0]
        b = bv[1]
        a0 = (a // 8) * 8
        nchunks = (b - a0 + CH - 1) // CH

        pltpu.sync_copy(zeros_hbm, staging)

        def chunk(c, carry2):
            nominal = a0 + c * CH
            base = jnp.minimum(nominal, nq - CH)
            pltpu.sync_copy(ids_hbm.at[pl.ds(base, CH)], idbuf)
            pltpu.sync_copy(xq_hbm.at[pl.ds(base * D, CH * D)], xbuf)

            def group(g, carry3):
                ids_v = idbuf[pl.ds(g * 16, 16)]
                grow = base + g * 16 + lane
                valid = ((ids_v >= winbase) & (ids_v < winbase + AW)
                         & (grow >= nominal))
                relm = jnp.where(valid, ids_v - winbase, AW) * D
                xbase = g * (16 * D) + laneD
                def phase(p, carry4):
                    rot = (lane + p) & 15
                    for kk in range(D // 16):
                        t = rot + kk * 16
                        v = plsc.load_gather(xbuf, [xbase + t])
                        plsc.addupdate_scatter(staging, [relm + t], v)
                    return carry4

                lax.fori_loop(0, 16, phase, 0)
                return carry3

            lax.fori_loop(0, CH // 16, group, 0)
            return carry2

        lax.fori_loop(0, nchunks, chunk, 0)
        pltpu.sync_copy(staging.at[pl.ds(0, AW * D)],
                        agg_hbm.at[pl.ds(winbase * D, AW * D)])
        return carry

    lax.fori_loop(0, NWT, window, 0)


def _sc_compiler_params():
    cp = pltpu.CompilerParams()
    if "needs_layout_passes" in pltpu.CompilerParams.__dataclass_fields__:
        cp = dataclasses.replace(cp, needs_layout_passes=False)
    return cp


def _tc_body(x_edge_ref, agg_ref, w_ref, out_ref):
    out_ref[...] = (
        jnp.dot(x_edge_ref[...], w_ref[0:D, :], preferred_element_type=jnp.float32)
        + jnp.dot(agg_ref[...], w_ref[D:2 * D, :], preferred_element_type=jnp.float32))


def kernel(x_q, x_edge, edge_ids, w_q):
    num_edges = x_edge.shape[0]
    ids32 = edge_ids.astype(jnp.int32)
    nwin = num_edges // AW
    win_edges = jnp.arange(0, num_edges + 1, AW, dtype=jnp.int32)
    bounds = jnp.searchsorted(ids32, win_edges).astype(jnp.int32)
    bounds = jnp.pad(bounds, (0, 15), mode="edge")
    zeros = jnp.zeros((RS * D,), jnp.float32)

    sc_agg = pl.kernel(
        _sc_agg_body,
        out_type=jax.ShapeDtypeStruct((num_edges * D,), jnp.float32),
        mesh=plsc.VectorSubcoreMesh(core_axis_name="c", subcore_axis_name="s"),
        scratch_types=[
            pltpu.VMEM((RS * D,), jnp.float32),
            pltpu.VMEM((CH * D,), jnp.float32),
            pltpu.VMEM((CH,), jnp.int32),
            pltpu.VMEM((nwin + 16,), jnp.int32),
        ],
        compiler_params=_sc_compiler_params(),
    )
    agg = sc_agg(x_q.reshape(-1), ids32, bounds, zeros).reshape(num_edges, D)

    grid = num_edges // E_BLK
    return pl.pallas_call(
        _tc_body,
        grid=(grid,),
        in_specs=[
            pl.BlockSpec((E_BLK, D), lambda i: (i, 0)),
            pl.BlockSpec((E_BLK, D), lambda i: (i, 0)),
            pl.BlockSpec((2 * D, D), lambda i: (0, 0)),
        ],
        out_specs=pl.BlockSpec((E_BLK, D), lambda i: (i, 0)),
        out_shape=jax.ShapeDtypeStruct((num_edges, D), jnp.float32),
        compiler_params=pltpu.CompilerParams(
            dimension_semantics=("arbitrary",)),
    )(x_edge, agg, w_q)
